# Initial kernel scaffold; baseline (speedup 1.0000x reference)
import jax
import jax.numpy as jnp
from jax.experimental import pallas as pl


def kernel(h, pos, edge_index, We1, be1, We2, be2, Wx1, bx1, Wx2, Wh1, bh1, Wh2, bh2):
    raise NotImplementedError("write your pallas kernel here")


# speedup vs baseline: 3.1369x; 3.1369x over previous
import functools

import jax
import jax.numpy as jnp
from jax import lax
from jax.experimental import pallas as pl
from jax.experimental.pallas import tpu as pltpu
from jax.experimental.pallas import tpu_sc as plsc

F32 = jnp.float32
I32 = jnp.int32


def _silu(x):
    return x * (1.0 / (1.0 + jnp.exp(-x)))


def _ceil_to(x, m):
    return ((x + m - 1) // m) * m


def kernel(h, pos, edge_index, We1, be1, We2, be2, Wx1, bx1, Wx2, Wh1, bh1, Wh2, bh2):
    N, D = h.shape
    P = pos.shape[1]
    E = edge_index.shape[1]
    H = We2.shape[0]
    PW = 4

    try:
        info = plsc.get_sparse_core_info()
        NC, NS = info.num_cores, info.num_subcores
    except Exception:
        NC, NS = 2, 16
    NW = NC * NS
    C = 128
    NQ = C // 16
    K = -(-E // (NW * C))
    EP = NW * C * K
    NP = _ceil_to(N + 1, 1024)
    RP = NP // NS

    hp = jnp.zeros((NP, D), F32).at[:N].set(h)
    pp = jnp.zeros((NP, PW), F32).at[:N, :P].set(pos)
    src = edge_index[0]
    dst = edge_index[1]
    pad = EP - E
    srcp = jnp.concatenate([src, jnp.zeros((pad,), I32)])
    dstp = jnp.concatenate([dst, jnp.full((pad,), N, I32)])
    src3 = srcp.reshape(NW, K, C)
    dst3 = dstp.reshape(NW, K, C)
    A = We1[:D]
    B = We1[D:2 * D]
    w257 = We1[2 * D:2 * D + 1]
    be1r = be1.reshape(1, H)
    be2r = be2.reshape(1, H)
    bx1r = bx1.reshape(1, H)
    wx2r = Wx2.reshape(1, H)
    Wh1a = Wh1[:D]
    Wh1b = Wh1[D:]
    bh1r = bh1.reshape(1, H)
    bh2r = bh2.reshape(1, D)
    zeros_m = jnp.zeros((NP, H), F32)
    zeros_4 = jnp.zeros((NP, PW), F32)

    RB1 = 1024

    def k1_body(hb, a_ref, b_ref, be1_ref, td, ts):
        hv = hb[...]
        td[...] = jnp.dot(hv, a_ref[...], preferred_element_type=F32) + be1_ref[...]
        ts[...] = jnp.dot(hv, b_ref[...], preferred_element_type=F32)

    Td, Ts = pl.pallas_call(
        k1_body,
        grid=(NP // RB1,),
        in_specs=[
            pl.BlockSpec((RB1, D), lambda i: (i, 0)),
            pl.BlockSpec((D, H), lambda i: (0, 0)),
            pl.BlockSpec((D, H), lambda i: (0, 0)),
            pl.BlockSpec((1, H), lambda i: (0, 0)),
        ],
        out_specs=[
            pl.BlockSpec((RB1, H), lambda i: (i, 0)),
            pl.BlockSpec((RB1, H), lambda i: (i, 0)),
        ],
        out_shape=[
            jax.ShapeDtypeStruct((NP, H), F32),
            jax.ShapeDtypeStruct((NP, H), F32),
        ],
    )(hp, A, B, be1r)

    mesh = plsc.VectorSubcoreMesh(core_axis_name="c", subcore_axis_name="s")

    @functools.partial(
        pl.kernel,
        out_type=(jax.ShapeDtypeStruct((EP, H), F32),
                  jax.ShapeDtypeStruct((EP, H), F32),
                  jax.ShapeDtypeStruct((EP * PW,), F32)),
        mesh=mesh,
        compiler_params=pltpu.CompilerParams(needs_layout_passes=False),
        scratch_types=[
            pltpu.VMEM((K, C), I32),
            pltpu.VMEM((K, C), I32),
            pltpu.VMEM((NP * PW,), F32),
            pltpu.VMEM((C, H), F32),
            pltpu.VMEM((C, H), F32),
            pltpu.VMEM((C * PW,), F32),
            pltpu.SemaphoreType.DMA,
            pltpu.SemaphoreType.DMA,
        ],
    )
    def k2(td_h, ts_h, dst_h, src_h, pos_h,
           gd_h, gs_h, rel_h,
           idxd, idxs, posv, bufd, bufs, relb, semd, sems):
        c = lax.axis_index("c")
        s = lax.axis_index("s")
        wid = s * NC + c
        pltpu.sync_copy(dst_h.at[wid], idxd)
        pltpu.sync_copy(src_h.at[wid], idxs)
        pltpu.sync_copy(pos_h, posv)
        base = wid * K
        lanes = lax.iota(I32, 16)

        def chunk(j, carry):
            cd = pltpu.async_copy(td_h.at[idxd.at[j]], bufd, semd)
            cs = pltpu.async_copy(ts_h.at[idxs.at[j]], bufs, sems)
            for q in range(NQ):
                dv = idxd[j, pl.ds(q * 16, 16)] * PW
                sv = idxs[j, pl.ds(q * 16, 16)] * PW
                rows = (lanes + (q * 16)) * PW
                d2 = jnp.zeros((16,), F32)
                for comp in range(P):
                    rc = (plsc.load_gather(posv, [dv + comp])
                          - plsc.load_gather(posv, [sv + comp]))
                    plsc.store_scatter(relb, [rows + comp], rc)
                    d2 = d2 + rc * rc
                plsc.store_scatter(relb, [rows + P], d2)
            cd.wait()
            cs.wait()
            row0 = (base + j) * C
            pltpu.sync_copy(bufd, gd_h.at[pl.ds(row0, C)])
            pltpu.sync_copy(bufs, gs_h.at[pl.ds(row0, C)])
            pltpu.sync_copy(relb, rel_h.at[pl.ds(row0 * PW, C * PW)])
            return carry

        lax.fori_loop(0, K, chunk, 0)

    Gd, Gs, Rel1 = k2(Td, Ts, dst3, src3, pp.reshape(NP * PW))
    Rel = Rel1.reshape(EP, PW)

    BE = 512

    def k3_body(gd, gs, rl, we2, be2_r, wx1, bx1_r, wx2_r, w257_r, evm, evt):
        relv = rl[...]
        d2 = relv[:, P:P + 1]
        pre = gd[...] + gs[...] + d2 * w257_r[...]
        m = _silu(pre)
        m = _silu(jnp.dot(m, we2[...], preferred_element_type=F32) + be2_r[...])
        t = _silu(jnp.dot(m, wx1[...], preferred_element_type=F32) + bx1_r[...])
        w = jnp.sum(t * wx2_r[...], axis=1, keepdims=True)
        evm[...] = m
        evt[...] = relv * w

    EVm, EVt = pl.pallas_call(
        k3_body,
        grid=(EP // BE,),
        in_specs=[
            pl.BlockSpec((BE, H), lambda i: (i, 0)),
            pl.BlockSpec((BE, H), lambda i: (i, 0)),
            pl.BlockSpec((BE, PW), lambda i: (i, 0)),
            pl.BlockSpec((H, H), lambda i: (0, 0)),
            pl.BlockSpec((1, H), lambda i: (0, 0)),
            pl.BlockSpec((H, H), lambda i: (0, 0)),
            pl.BlockSpec((1, H), lambda i: (0, 0)),
            pl.BlockSpec((1, H), lambda i: (0, 0)),
            pl.BlockSpec((1, H), lambda i: (0, 0)),
        ],
        out_specs=[
            pl.BlockSpec((BE, H), lambda i: (i, 0)),
            pl.BlockSpec((BE, PW), lambda i: (i, 0)),
        ],
        out_shape=[
            jax.ShapeDtypeStruct((EP, H), F32),
            jax.ShapeDtypeStruct((EP, PW), F32),
        ],
    )(Gd, Gs, Rel, We2, be2r, Wx1, bx1r, wx2r, w257)

    @functools.partial(
        pl.kernel,
        out_type=(jax.ShapeDtypeStruct((NP, H), F32),
                  jax.ShapeDtypeStruct((NP, H), F32)),
        mesh=mesh,
        compiler_params=pltpu.CompilerParams(needs_layout_passes=False),
        scratch_types=[
            pltpu.VMEM((K, C), I32),
            pltpu.VMEM((C, H), F32),
            pltpu.VMEM_SHARED((NP, H), F32),
            pltpu.SemaphoreType.DMA,
        ],
    )
    def k4a(evm_h, dst_h, zm_h, p0_h, p1_h, idxd, mbuf, accm, sem):
        c = lax.axis_index("c")
        s = lax.axis_index("s")
        wid = s * NC + c
        r0 = s * RP
        pltpu.sync_copy(zm_h.at[pl.ds(r0, RP)], accm.at[pl.ds(r0, RP)])
        pltpu.sync_copy(dst_h.at[wid], idxd)
        plsc.subcore_barrier()
        base = wid * K

        def chunk(j, carry):
            row0 = (base + j) * C
            pltpu.sync_copy(evm_h.at[pl.ds(row0, C)], mbuf)
            pltpu.sync_copy(mbuf, accm.at[idxd.at[j]], add=True)
            return carry

        lax.fori_loop(0, K, chunk, 0)
        plsc.subcore_barrier()

        @pl.when(c == 0)
        def _():
            pltpu.sync_copy(accm.at[pl.ds(r0, RP)], p0_h.at[pl.ds(r0, RP)])

        @pl.when(c == 1)
        def _():
            pltpu.sync_copy(accm.at[pl.ds(r0, RP)], p1_h.at[pl.ds(r0, RP)])

    P0, P1 = k4a(EVm, dst3, zeros_m)

    @functools.partial(
        pl.kernel,
        out_type=jax.ShapeDtypeStruct((NW, NP * PW), F32),
        mesh=mesh,
        compiler_params=pltpu.CompilerParams(needs_layout_passes=False),
        scratch_types=[
            pltpu.VMEM((K, C), I32),
            pltpu.VMEM((C * PW,), F32),
            pltpu.VMEM((NP * PW,), F32),
        ],
    )
    def k4b(evt_h, dst_h, z4_h, t4_h, idxd, tbuf, acc4):
        c = lax.axis_index("c")
        s = lax.axis_index("s")
        wid = s * NC + c
        pltpu.sync_copy(z4_h, acc4)
        pltpu.sync_copy(dst_h.at[wid], idxd)
        base = wid * K
        lanes = lax.iota(I32, 16)
        ones = jnp.full((16,), 1.0, F32)

        def chunk(j, carry):
            row0 = (base + j) * C
            pltpu.sync_copy(evt_h.at[pl.ds(row0 * PW, C * PW)], tbuf)
            for q in range(NQ):
                dv = idxd[j, pl.ds(q * 16, 16)] * PW
                rows = (lanes + (q * 16)) * PW
                for comp in range(P):
                    tv = plsc.load_gather(tbuf, [rows + comp])
                    plsc.addupdate_scatter(acc4, [dv + comp], tv)
                plsc.addupdate_scatter(acc4, [dv + P], ones)
            return carry

        lax.fori_loop(0, K, chunk, 0)
        pltpu.sync_copy(acc4, t4_h.at[wid])

    T4f = k4b(EVt.reshape(EP * PW), dst3, zeros_4.reshape(NP * PW))
    T4 = T4f.reshape(NW, NP, PW)

    RB5 = 1000

    def k5_body(hb, pb, p0, p1, t4b, wh1a, wh1b, bh1_r, wh2, bh2_r, ho, po):
        aggm = p0[...] + p1[...]
        t4 = jnp.sum(t4b[...], axis=0)
        inv = 1.0 / jnp.maximum(t4[:, P:P + 1], 1.0)
        aggm = aggm * inv
        hv = hb[...]
        u = _silu(jnp.dot(hv, wh1a[...], preferred_element_type=F32)
                  + jnp.dot(aggm, wh1b[...], preferred_element_type=F32)
                  + bh1_r[...])
        ho[...] = hv + jnp.dot(u, wh2[...], preferred_element_type=F32) + bh2_r[...]
        po[...] = pb[...] + t4 * inv

    h_out, pos4 = pl.pallas_call(
        k5_body,
        grid=(N // RB5,),
        in_specs=[
            pl.BlockSpec((RB5, D), lambda i: (i, 0)),
            pl.BlockSpec((RB5, PW), lambda i: (i, 0)),
            pl.BlockSpec((RB5, H), lambda i: (i, 0)),
            pl.BlockSpec((RB5, H), lambda i: (i, 0)),
            pl.BlockSpec((NW, RB5, PW), lambda i: (0, i, 0)),
            pl.BlockSpec((D, H), lambda i: (0, 0)),
            pl.BlockSpec((H, H), lambda i: (0, 0)),
            pl.BlockSpec((1, H), lambda i: (0, 0)),
            pl.BlockSpec((H, D), lambda i: (0, 0)),
            pl.BlockSpec((1, D), lambda i: (0, 0)),
        ],
        out_specs=[
            pl.BlockSpec((RB5, D), lambda i: (i, 0)),
            pl.BlockSpec((RB5, PW), lambda i: (i, 0)),
        ],
        out_shape=[
            jax.ShapeDtypeStruct((N, D), F32),
            jax.ShapeDtypeStruct((N, PW), F32),
        ],
    )(hp, pp, P0, P1, T4, Wh1a, Wh1b, bh1r, Wh2, bh2r)

    return (h_out, pos4[:, :P])


# speedup vs baseline: 3.5472x; 1.1308x over previous
import functools

import jax
import jax.numpy as jnp
from jax import lax
from jax.experimental import pallas as pl
from jax.experimental.pallas import tpu as pltpu
from jax.experimental.pallas import tpu_sc as plsc

F32 = jnp.float32
I32 = jnp.int32


def _silu(x):
    return x * (1.0 / (1.0 + jnp.exp(-x)))


def _ceil_to(x, m):
    return ((x + m - 1) // m) * m


def kernel(h, pos, edge_index, We1, be1, We2, be2, Wx1, bx1, Wx2, Wh1, bh1, Wh2, bh2):
    N, D = h.shape
    P = pos.shape[1]
    E = edge_index.shape[1]
    H = We2.shape[0]
    PW = 4

    try:
        info = plsc.get_sparse_core_info()
        NC, NS = info.num_cores, info.num_subcores
    except Exception:
        NC, NS = 2, 16
    NW = NC * NS
    C = 128
    NQ = C // 16
    K = -(-E // (NW * C))
    KH = (K + 1) // 2
    EP = NW * C * K
    NP = _ceil_to(N + 1, NS * 8)
    RP = NP // NS

    pp = jnp.zeros((NP, PW), F32).at[:N, :P].set(pos)
    src = edge_index[0]
    dst = edge_index[1]
    pad = EP - E
    srcp = jnp.concatenate([src, jnp.zeros((pad,), I32)])
    dstp = jnp.concatenate([dst, jnp.full((pad,), N, I32)])
    src3 = srcp.reshape(NW, K, C)
    dst3 = dstp.reshape(NW, K, C)
    A = We1[:D]
    B = We1[D:2 * D]
    w257 = We1[2 * D:2 * D + 1]
    be1r = be1.reshape(1, H)
    be2r = be2.reshape(1, H)
    bx1r = bx1.reshape(1, H)
    wx2r = Wx2.reshape(1, H)
    Wh1a = Wh1[:D]
    Wh1b = Wh1[D:]
    bh1r = bh1.reshape(1, H)
    bh2r = bh2.reshape(1, D)
    zeros_m = jnp.zeros((NP, H), F32)
    zeros_4 = jnp.zeros((NP * PW,), F32)

    RB1 = 1000

    def k1_body(hb, a_ref, b_ref, be1_ref, td, ts):
        hv = hb[...]
        td[...] = jnp.dot(hv, a_ref[...], preferred_element_type=F32) + be1_ref[...]
        ts[...] = jnp.dot(hv, b_ref[...], preferred_element_type=F32)

    Td, Ts = pl.pallas_call(
        k1_body,
        grid=(N // RB1,),
        in_specs=[
            pl.BlockSpec((RB1, D), lambda i: (i, 0)),
            pl.BlockSpec((D, H), lambda i: (0, 0)),
            pl.BlockSpec((D, H), lambda i: (0, 0)),
            pl.BlockSpec((1, H), lambda i: (0, 0)),
        ],
        out_specs=[
            pl.BlockSpec((RB1, H), lambda i: (i, 0)),
            pl.BlockSpec((RB1, H), lambda i: (i, 0)),
        ],
        out_shape=[
            jax.ShapeDtypeStruct((NP, H), F32),
            jax.ShapeDtypeStruct((NP, H), F32),
        ],
    )(h, A, B, be1r)

    mesh = plsc.VectorSubcoreMesh(core_axis_name="c", subcore_axis_name="s")

    @functools.partial(
        pl.kernel,
        out_type=(jax.ShapeDtypeStruct((EP, H), F32),
                  jax.ShapeDtypeStruct((EP * PW,), F32)),
        mesh=mesh,
        compiler_params=pltpu.CompilerParams(needs_layout_passes=False),
        scratch_types=[
            pltpu.VMEM((K, C), I32),
            pltpu.VMEM((K, C), I32),
            pltpu.VMEM((NP * PW,), F32),
            pltpu.VMEM((C, H), F32),
            pltpu.VMEM((C, H), F32),
            pltpu.VMEM((C * PW,), F32),
            pltpu.VMEM((C * PW,), F32),
            pltpu.SemaphoreType.DMA,
            pltpu.SemaphoreType.DMA,
            pltpu.SemaphoreType.DMA,
            pltpu.SemaphoreType.DMA,
            pltpu.SemaphoreType.DMA,
            pltpu.SemaphoreType.DMA,
        ],
    )
    def k2(td_h, ts_h, dst_h, src_h, pos_h,
           g_h, rel_h,
           idxd, idxs, posv, buf0, buf1, relb0, relb1,
           semgd0, semgd1, semgs0, semgs1, semw0, semw1):
        c = lax.axis_index("c")
        s = lax.axis_index("s")
        wid = s * NC + c
        pltpu.sync_copy(dst_h.at[wid], idxd)
        pltpu.sync_copy(src_h.at[wid], idxs)
        pltpu.sync_copy(pos_h, posv)
        base = wid * K
        lanes = lax.iota(I32, 16)

        def rel_compute(j, relb):
            for q in range(NQ):
                dv = idxd[j, pl.ds(q * 16, 16)] * PW
                sv = idxs[j, pl.ds(q * 16, 16)] * PW
                rows = (lanes + (q * 16)) * PW
                d2 = jnp.zeros((16,), F32)
                for comp in range(P):
                    rc = (plsc.load_gather(posv, [dv + comp])
                          - plsc.load_gather(posv, [sv + comp]))
                    plsc.store_scatter(relb, [rows + comp], rc)
                    d2 = d2 + rc * rc
                plsc.store_scatter(relb, [rows + P], d2)

        def issue_gd(j, buf, sem):
            pltpu.async_copy(td_h.at[idxd.at[j]], buf, sem)

        def wait_gd(j, buf, sem):
            pltpu.make_async_copy(td_h.at[idxd.at[j]], buf, sem).wait()

        def issue_gs(j, buf, sem):
            pltpu.async_copy(ts_h.at[idxs.at[j]], buf, sem, add=True)

        def wait_gs(j, buf, sem):
            pltpu.make_async_copy(ts_h.at[idxs.at[j]], buf, sem).wait()

        def issue_wr(j, buf, relb, sem):
            row0 = (base + j) * C
            pltpu.async_copy(buf, g_h.at[pl.ds(row0, C)], sem)
            pltpu.async_copy(relb, rel_h.at[pl.ds(row0 * PW, C * PW)], sem)

        def drain_wr(buf, relb, sem):
            row0 = base * C
            pltpu.make_async_copy(buf, g_h.at[pl.ds(row0, C)], sem).wait()
            pltpu.make_async_copy(relb, rel_h.at[pl.ds(row0 * PW, C * PW)], sem).wait()

        issue_gd(0, buf0, semgd0)

        def body(jj, carry):
            j0 = 2 * jj
            j1 = j0 + 1
            j2 = j0 + 2
            rel_compute(j0, relb0)
            wait_gd(j0, buf0, semgd0)
            issue_gs(j0, buf0, semgs0)

            @pl.when(j1 < K)
            def _():
                @pl.when(jj > 0)
                def _():
                    drain_wr(buf1, relb1, semw1)
                issue_gd(j1, buf1, semgd1)

            wait_gs(j0, buf0, semgs0)
            issue_wr(j0, buf0, relb0, semw0)

            @pl.when(j1 < K)
            def _():
                rel_compute(j1, relb1)
                wait_gd(j1, buf1, semgd1)
                issue_gs(j1, buf1, semgs1)

                @pl.when(j2 < K)
                def _():
                    drain_wr(buf0, relb0, semw0)
                    issue_gd(j2, buf0, semgd0)

                wait_gs(j1, buf1, semgs1)
                issue_wr(j1, buf1, relb1, semw1)

            return carry

        lax.fori_loop(0, KH, body, 0)
        drain_wr(buf0, relb0, semw0)
        if K >= 2:
            drain_wr(buf1, relb1, semw1)

    G, Rel1 = k2(Td, Ts, dst3, src3, pp.reshape(NP * PW))
    Rel = Rel1.reshape(EP, PW)

    BE = 512

    def k3_body(g, rl, we2, be2_r, wx1, bx1_r, wx2_r, w257_r, evm, evt):
        relv = rl[...]
        d2 = relv[:, P:P + 1]
        pre = g[...] + d2 * w257_r[...]
        m = _silu(pre)
        m = _silu(jnp.dot(m, we2[...], preferred_element_type=F32) + be2_r[...])
        t = _silu(jnp.dot(m, wx1[...], preferred_element_type=F32) + bx1_r[...])
        w = jnp.sum(t * wx2_r[...], axis=1, keepdims=True)
        evm[...] = m
        evt[...] = relv * w

    EVm, EVt = pl.pallas_call(
        k3_body,
        grid=(EP // BE,),
        in_specs=[
            pl.BlockSpec((BE, H), lambda i: (i, 0)),
            pl.BlockSpec((BE, PW), lambda i: (i, 0)),
            pl.BlockSpec((H, H), lambda i: (0, 0)),
            pl.BlockSpec((1, H), lambda i: (0, 0)),
            pl.BlockSpec((H, H), lambda i: (0, 0)),
            pl.BlockSpec((1, H), lambda i: (0, 0)),
            pl.BlockSpec((1, H), lambda i: (0, 0)),
            pl.BlockSpec((1, H), lambda i: (0, 0)),
        ],
        out_specs=[
            pl.BlockSpec((BE, H), lambda i: (i, 0)),
            pl.BlockSpec((BE, PW), lambda i: (i, 0)),
        ],
        out_shape=[
            jax.ShapeDtypeStruct((EP, H), F32),
            jax.ShapeDtypeStruct((EP, PW), F32),
        ],
    )(G, Rel, We2, be2r, Wx1, bx1r, wx2r, w257)

    @functools.partial(
        pl.kernel,
        out_type=(jax.ShapeDtypeStruct((NP, H), F32),
                  jax.ShapeDtypeStruct((NP, H), F32)),
        mesh=mesh,
        compiler_params=pltpu.CompilerParams(needs_layout_passes=False),
        scratch_types=[
            pltpu.VMEM((K, C), I32),
            pltpu.VMEM((C, H), F32),
            pltpu.VMEM((C, H), F32),
            pltpu.VMEM_SHARED((NP, H), F32),
            pltpu.SemaphoreType.DMA,
            pltpu.SemaphoreType.DMA,
        ],
    )
    def k4a(evm_h, dst_h, zm_h, p0_h, p1_h, idxd, m0, m1, accm, semld0, semld1):
        c = lax.axis_index("c")
        s = lax.axis_index("s")
        wid = s * NC + c
        r0 = s * RP
        pltpu.sync_copy(zm_h.at[pl.ds(r0, RP)], accm.at[pl.ds(r0, RP)])
        pltpu.sync_copy(dst_h.at[wid], idxd)
        plsc.subcore_barrier()
        base = wid * K

        def issue_ld(j, buf, sem):
            pltpu.async_copy(evm_h.at[pl.ds((base + j) * C, C)], buf, sem)

        def wait_ld(j, buf, sem):
            pltpu.make_async_copy(evm_h.at[pl.ds((base + j) * C, C)], buf, sem).wait()

        issue_ld(0, m0, semld0)

        def body(jj, carry):
            j0 = 2 * jj
            j1 = j0 + 1
            j2 = j0 + 2
            wait_ld(j0, m0, semld0)

            @pl.when(j1 < K)
            def _():
                issue_ld(j1, m1, semld1)

            pltpu.sync_copy(m0, accm.at[idxd.at[j0]], add=True)

            @pl.when(j1 < K)
            def _():
                wait_ld(j1, m1, semld1)

                @pl.when(j2 < K)
                def _():
                    issue_ld(j2, m0, semld0)

                pltpu.sync_copy(m1, accm.at[idxd.at[j1]], add=True)

            return carry

        lax.fori_loop(0, KH, body, 0)
        plsc.subcore_barrier()

        @pl.when(c == 0)
        def _():
            pltpu.sync_copy(accm.at[pl.ds(r0, RP)], p0_h.at[pl.ds(r0, RP)])

        @pl.when(c == 1)
        def _():
            pltpu.sync_copy(accm.at[pl.ds(r0, RP)], p1_h.at[pl.ds(r0, RP)])

    P0, P1 = k4a(EVm, dst3, zeros_m)

    @functools.partial(
        pl.kernel,
        out_type=jax.ShapeDtypeStruct((NW, NP * PW), F32),
        mesh=mesh,
        compiler_params=pltpu.CompilerParams(needs_layout_passes=False),
        scratch_types=[
            pltpu.VMEM((K, C), I32),
            pltpu.VMEM((C * PW,), F32),
            pltpu.VMEM((C * PW,), F32),
            pltpu.VMEM((NP * PW,), F32),
            pltpu.SemaphoreType.DMA,
            pltpu.SemaphoreType.DMA,
        ],
    )
    def k4b(evt_h, dst_h, z4_h, t4_h, idxd, t0, t1, acc4, semld0, semld1):
        c = lax.axis_index("c")
        s = lax.axis_index("s")
        wid = s * NC + c
        pltpu.sync_copy(z4_h, acc4)
        pltpu.sync_copy(dst_h.at[wid], idxd)
        base = wid * K
        lanes = lax.iota(I32, 16)
        ones = jnp.full((16,), 1.0, F32)

        def issue_ld(j, buf, sem):
            pltpu.async_copy(evt_h.at[pl.ds((base + j) * C * PW, C * PW)], buf, sem)

        def wait_ld(j, buf, sem):
            pltpu.make_async_copy(
                evt_h.at[pl.ds((base + j) * C * PW, C * PW)], buf, sem).wait()

        def scat(j, tbuf):
            for q in range(NQ):
                dv = idxd[j, pl.ds(q * 16, 16)] * PW
                rows = (lanes + (q * 16)) * PW
                for comp in range(P):
                    tv = plsc.load_gather(tbuf, [rows + comp])
                    plsc.addupdate_scatter(acc4, [dv + comp], tv)
                plsc.addupdate_scatter(acc4, [dv + P], ones)

        issue_ld(0, t0, semld0)

        def body(jj, carry):
            j0 = 2 * jj
            j1 = j0 + 1
            j2 = j0 + 2
            wait_ld(j0, t0, semld0)

            @pl.when(j1 < K)
            def _():
                issue_ld(j1, t1, semld1)

            scat(j0, t0)

            @pl.when(j1 < K)
            def _():
                wait_ld(j1, t1, semld1)

                @pl.when(j2 < K)
                def _():
                    issue_ld(j2, t0, semld0)

                scat(j1, t1)

            return carry

        lax.fori_loop(0, KH, body, 0)
        pltpu.sync_copy(acc4, t4_h.at[wid])

    T4f = k4b(EVt.reshape(EP * PW), dst3, zeros_4)
    T4 = T4f.reshape(NW, NP, PW)

    RB5 = 1000

    def k5_body(hb, pb, p0, p1, t4b, wh1a, wh1b, bh1_r, wh2, bh2_r, ho, po):
        aggm = p0[...] + p1[...]
        t4 = jnp.sum(t4b[...], axis=0)
        inv = 1.0 / jnp.maximum(t4[:, P:P + 1], 1.0)
        aggm = aggm * inv
        hv = hb[...]
        u = _silu(jnp.dot(hv, wh1a[...], preferred_element_type=F32)
                  + jnp.dot(aggm, wh1b[...], preferred_element_type=F32)
                  + bh1_r[...])
        ho[...] = hv + jnp.dot(u, wh2[...], preferred_element_type=F32) + bh2_r[...]
        po[...] = pb[...] + t4 * inv

    h_out, pos4 = pl.pallas_call(
        k5_body,
        grid=(N // RB5,),
        in_specs=[
            pl.BlockSpec((RB5, D), lambda i: (i, 0)),
            pl.BlockSpec((RB5, PW), lambda i: (i, 0)),
            pl.BlockSpec((RB5, H), lambda i: (i, 0)),
            pl.BlockSpec((RB5, H), lambda i: (i, 0)),
            pl.BlockSpec((NW, RB5, PW), lambda i: (0, i, 0)),
            pl.BlockSpec((D, H), lambda i: (0, 0)),
            pl.BlockSpec((H, H), lambda i: (0, 0)),
            pl.BlockSpec((1, H), lambda i: (0, 0)),
            pl.BlockSpec((H, D), lambda i: (0, 0)),
            pl.BlockSpec((1, D), lambda i: (0, 0)),
        ],
        out_specs=[
            pl.BlockSpec((RB5, D), lambda i: (i, 0)),
            pl.BlockSpec((RB5, PW), lambda i: (i, 0)),
        ],
        out_shape=[
            jax.ShapeDtypeStruct((N, D), F32),
            jax.ShapeDtypeStruct((N, PW), F32),
        ],
    )(h, pp, P0, P1, T4, Wh1a, Wh1b, bh1r, Wh2, bh2r)

    return (h_out, pos4[:, :P])
